# Initial kernel scaffold; baseline (speedup 1.0000x reference)
import jax
import jax.numpy as jnp
from jax.experimental import pallas as pl


def kernel(z, emb_w, conv_w, conv_b, means_v, stds_v):
    raise NotImplementedError("write your pallas kernel here")


# speedup vs baseline: 1.4336x; 1.4336x over previous
import jax
import jax.numpy as jnp
from jax.experimental import pallas as pl
from jax.experimental.pallas import tpu as pltpu

N_E = 256
E_DIM = 1024
B = 16
C = 512
HW = 1024
ROWS_PER_B = 512
N_ROWS = B * ROWS_PER_B
N_ELEM = float(B * C * HW)


def _body(zf_ref, znat_ref, e_ref, cw_ref, cb_ref,
          out_ref, idx_ref, enc_ref, loss_ref, perp_ref,
          hist_ref, lacc_ref):
    b = pl.program_id(0)
    zfb = zf_ref[0]
    es = e_ref[...]
    zn = jnp.sum(zfb * zfb, axis=1, keepdims=True)
    en = jnp.sum(es * es, axis=1)
    m = jax.lax.dot_general(zfb, es, (((1,), (1,)), ((), ())),
                            preferred_element_type=jnp.float32)
    d = zn + en[None, :] - 2.0 * m
    dmin = jnp.min(d, axis=1, keepdims=True)
    kio = jax.lax.broadcasted_iota(jnp.int32, (ROWS_PER_B, N_E), 1)
    idx = jnp.min(jnp.where(d == dmin, kio, N_E), axis=1)
    idx_ref[0, 0, :] = idx
    oh = (kio == idx[:, None]).astype(jnp.float32)
    enc_ref[0] = oh
    zq = jax.lax.dot_general(oh, es, (((1,), (0,)), ((), ())),
                             preferred_element_type=jnp.float32)
    diff = zq - znat_ref[0]
    lp = jnp.sum(diff * diff)
    h = jnp.sum(oh, axis=0)
    lacc_ref[0, 0] = jnp.where(b == 0, lp, lacc_ref[0, 0] + lp)
    hist_ref[0] = jnp.where(b == 0, h, hist_ref[0] + h)
    outb = jax.lax.dot_general(cw_ref[...], zq, (((1,), (0,)), ((), ())),
                               preferred_element_type=jnp.float32)
    out_ref[0] = outb + cb_ref[0][:, None]

    @pl.when(b == B - 1)
    def _():
        loss_ref[...] = jnp.full((1, 1), 1.25 * lacc_ref[0, 0] / N_ELEM,
                                 jnp.float32)
        em = hist_ref[0] / float(N_ROWS)
        perp = jnp.exp(-jnp.sum(em * jnp.log(em + 1e-10)))
        perp_ref[...] = jnp.full((1, 1), perp, jnp.float32)


def kernel(z, emb_w, conv_w, conv_b, means_v, stds_v):
    sg = jax.lax.stop_gradient
    noise = jax.random.normal(jax.random.key(42), (), dtype=jnp.float32)
    std = sg(jnp.abs(stds_v)) + sg(noise)
    mean = sg(jnp.mean(means_v))
    es = emb_w * std + mean
    zf = jnp.transpose(z, (0, 2, 3, 1)).reshape(B, ROWS_PER_B, E_DIM)
    znat = z.reshape(B, ROWS_PER_B, E_DIM)
    cb2 = conv_b.reshape(1, C)

    grid = (B,)
    out, idx3, enc, loss2, perp2 = pl.pallas_call(
        _body,
        grid=grid,
        in_specs=[
            pl.BlockSpec((1, ROWS_PER_B, E_DIM), lambda b: (b, 0, 0)),
            pl.BlockSpec((1, ROWS_PER_B, E_DIM), lambda b: (b, 0, 0)),
            pl.BlockSpec((N_E, E_DIM), lambda b: (0, 0)),
            pl.BlockSpec((C, C), lambda b: (0, 0)),
            pl.BlockSpec((1, C), lambda b: (0, 0)),
        ],
        out_specs=[
            pl.BlockSpec((1, ROWS_PER_B, E_DIM), lambda b: (b, 0, 0)),
            pl.BlockSpec((1, 1, ROWS_PER_B), lambda b: (b, 0, 0)),
            pl.BlockSpec((1, ROWS_PER_B, N_E), lambda b: (b, 0, 0)),
            pl.BlockSpec((1, 1), lambda b: (0, 0)),
            pl.BlockSpec((1, 1), lambda b: (0, 0)),
        ],
        out_shape=[
            jax.ShapeDtypeStruct((B, ROWS_PER_B, E_DIM), jnp.float32),
            jax.ShapeDtypeStruct((B, 1, ROWS_PER_B), jnp.int32),
            jax.ShapeDtypeStruct((B, ROWS_PER_B, N_E), jnp.float32),
            jax.ShapeDtypeStruct((1, 1), jnp.float32),
            jax.ShapeDtypeStruct((1, 1), jnp.float32),
        ],
        scratch_shapes=[
            pltpu.VMEM((1, N_E), jnp.float32),
            pltpu.SMEM((1, 1), jnp.float32),
        ],
    )(zf, znat, es, conv_w, cb2)

    out = out.reshape(z.shape)
    loss = loss2.reshape(())
    perplexity = perp2.reshape(())
    min_encodings = enc.reshape(N_ROWS, N_E)
    min_encoding_indices = idx3.reshape(N_ROWS, 1)
    return (out, loss, (perplexity, min_encodings, min_encoding_indices))


# speedup vs baseline: 1.7273x; 1.2049x over previous
import jax
import jax.numpy as jnp
from jax.experimental import pallas as pl
from jax.experimental.pallas import tpu as pltpu

N_E = 256
E_DIM = 1024
B = 16
C = 512
HW = 1024
ROWS_PER_B = 512
N_ROWS = B * ROWS_PER_B
N_ELEM = float(B * C * HW)


def _body(znat_ref, e_ref, cw_ref, cb_ref,
          out_ref, idx_ref, enc_ref, loss_ref, perp_ref,
          hist_ref, lacc_ref):
    b = pl.program_id(0)
    znb = znat_ref[0]
    zfb = znb.T.reshape(ROWS_PER_B, E_DIM)
    es = e_ref[...]
    zn = jnp.sum(zfb * zfb, axis=1, keepdims=True)
    en = jnp.sum(es * es, axis=1)
    m = jax.lax.dot_general(zfb, es, (((1,), (1,)), ((), ())),
                            preferred_element_type=jnp.float32)
    d = zn + en[None, :] - 2.0 * m
    dmin = jnp.min(d, axis=1, keepdims=True)
    kio = jax.lax.broadcasted_iota(jnp.int32, (ROWS_PER_B, N_E), 1)
    idx = jnp.min(jnp.where(d == dmin, kio, N_E), axis=1)
    idx_ref[0, 0, :] = idx
    oh = (kio == idx[:, None]).astype(jnp.float32)
    enc_ref[0] = oh
    zq = jax.lax.dot_general(oh, es, (((1,), (0,)), ((), ())),
                             preferred_element_type=jnp.float32)
    diff = zq - znb
    lp = jnp.sum(diff * diff)
    h = jnp.sum(oh, axis=0)
    lacc_ref[0, 0] = jnp.where(b == 0, lp, lacc_ref[0, 0] + lp)
    hist_ref[0] = jnp.where(b == 0, h, hist_ref[0] + h)
    outb = jax.lax.dot_general(cw_ref[...], zq, (((1,), (0,)), ((), ())),
                               preferred_element_type=jnp.float32)
    out_ref[0] = outb + cb_ref[0][:, None]

    @pl.when(b == B - 1)
    def _():
        loss_ref[...] = jnp.full((1, 1), 1.25 * lacc_ref[0, 0] / N_ELEM,
                                 jnp.float32)
        em = hist_ref[0] / float(N_ROWS)
        perp = jnp.exp(-jnp.sum(em * jnp.log(em + 1e-10)))
        perp_ref[...] = jnp.full((1, 1), perp, jnp.float32)


def kernel(z, emb_w, conv_w, conv_b, means_v, stds_v):
    sg = jax.lax.stop_gradient
    noise = jax.random.normal(jax.random.key(42), (), dtype=jnp.float32)
    std = sg(jnp.abs(stds_v)) + sg(noise)
    mean = sg(jnp.mean(means_v))
    es = emb_w * std + mean
    znat = z.reshape(B, ROWS_PER_B, E_DIM)
    cb2 = conv_b.reshape(1, C)

    grid = (B,)
    out, idx3, enc, loss2, perp2 = pl.pallas_call(
        _body,
        grid=grid,
        in_specs=[
            pl.BlockSpec((1, ROWS_PER_B, E_DIM), lambda b: (b, 0, 0)),
            pl.BlockSpec((N_E, E_DIM), lambda b: (0, 0)),
            pl.BlockSpec((C, C), lambda b: (0, 0)),
            pl.BlockSpec((1, C), lambda b: (0, 0)),
        ],
        out_specs=[
            pl.BlockSpec((1, ROWS_PER_B, E_DIM), lambda b: (b, 0, 0)),
            pl.BlockSpec((1, 1, ROWS_PER_B), lambda b: (b, 0, 0)),
            pl.BlockSpec((1, ROWS_PER_B, N_E), lambda b: (b, 0, 0)),
            pl.BlockSpec((1, 1), lambda b: (0, 0)),
            pl.BlockSpec((1, 1), lambda b: (0, 0)),
        ],
        out_shape=[
            jax.ShapeDtypeStruct((B, ROWS_PER_B, E_DIM), jnp.float32),
            jax.ShapeDtypeStruct((B, 1, ROWS_PER_B), jnp.int32),
            jax.ShapeDtypeStruct((B, ROWS_PER_B, N_E), jnp.float32),
            jax.ShapeDtypeStruct((1, 1), jnp.float32),
            jax.ShapeDtypeStruct((1, 1), jnp.float32),
        ],
        scratch_shapes=[
            pltpu.VMEM((1, N_E), jnp.float32),
            pltpu.SMEM((1, 1), jnp.float32),
        ],
    )(znat, es, conv_w, cb2)

    out = out.reshape(z.shape)
    loss = loss2.reshape(())
    perplexity = perp2.reshape(())
    min_encodings = enc.reshape(N_ROWS, N_E)
    min_encoding_indices = idx3.reshape(N_ROWS, 1)
    return (out, loss, (perplexity, min_encodings, min_encoding_indices))


# speedup vs baseline: 4.2438x; 2.4569x over previous
import jax
import jax.numpy as jnp
from jax.experimental import pallas as pl
from jax.experimental.pallas import tpu as pltpu

N_E = 256
E_DIM = 1024
B = 16
C = 512
HW = 1024
ROWS_PER_B = 512
N_ROWS = B * ROWS_PER_B
N_ELEM = float(B * C * HW)


def _body(zt_ref, emb_ref, sm_ref, cw_ref, cb_ref,
          out_ref, idx_ref, enc_ref, loss_ref, perp_ref,
          hist_ref, lacc_ref):
    b = pl.program_id(0)
    ztb = zt_ref[0]
    zfb = ztb.reshape(ROWS_PER_B, E_DIM)
    znbT = ztb.reshape(HW, C)
    es = emb_ref[...] * sm_ref[0, 0] + sm_ref[0, 1]
    esT = es.T
    zn = jnp.sum(zfb * zfb, axis=1, keepdims=True)
    znT = zn.T
    en = jnp.sum(es * es, axis=1, keepdims=True)
    mT = jax.lax.dot_general(es, zfb, (((1,), (1,)), ((), ())),
                             preferred_element_type=jnp.float32)
    dT = znT + en - 2.0 * mT
    dminT = jnp.min(dT, axis=0, keepdims=True)
    kioS = jax.lax.broadcasted_iota(jnp.int32, (N_E, ROWS_PER_B), 0)
    idxL = jnp.min(jnp.where(dT == dminT, kioS, N_E), axis=0,
                   keepdims=True)
    idx_ref[0] = idxL
    idxS = idxL.T
    kioL = jax.lax.broadcasted_iota(jnp.int32, (ROWS_PER_B, N_E), 1)
    oh = (kioL == idxS).astype(jnp.float32)
    enc_ref[0] = oh
    ohT = (kioS == idxL).astype(jnp.float32)
    zqT = jax.lax.dot_general(esT, ohT, (((1,), (0,)), ((), ())),
                              preferred_element_type=jnp.float32)
    diff = zqT - znbT
    lp = jnp.sum(diff * diff)
    h = jnp.sum(oh, axis=0)
    lacc_ref[0, 0] = jnp.where(b == 0, lp, lacc_ref[0, 0] + lp)
    hist_ref[0] = jnp.where(b == 0, h, hist_ref[0] + h)
    ocw = jax.lax.dot_general(ohT, cw_ref[...], (((1,), (1,)), ((), ())),
                              preferred_element_type=jnp.float32)
    outT = jax.lax.dot_general(esT, ocw, (((1,), (0,)), ((), ())),
                               preferred_element_type=jnp.float32)
    out_ref[0] = (outT + cb_ref[0][None, :]).reshape(32, 32, C)

    @pl.when(b == B - 1)
    def _():
        loss_ref[...] = jnp.full((1, 1), 1.25 * lacc_ref[0, 0] / N_ELEM,
                                 jnp.float32)
        em = hist_ref[0] / float(N_ROWS)
        perp = jnp.exp(-jnp.sum(em * jnp.log(em + 1e-10)))
        perp_ref[...] = jnp.full((1, 1), perp, jnp.float32)


def kernel(z, emb_w, conv_w, conv_b, means_v, stds_v):
    sg = jax.lax.stop_gradient
    noise = jax.random.normal(jax.random.key(42), (), dtype=jnp.float32)
    std = sg(jnp.abs(stds_v)) + sg(noise)
    mean = sg(jnp.mean(means_v))
    sm = jnp.stack([std, mean]).reshape(1, 2)
    zt = jnp.transpose(z, (0, 2, 3, 1))
    cb2 = conv_b.reshape(1, C)

    grid = (B,)
    out4, idx3, enc, loss2, perp2 = pl.pallas_call(
        _body,
        grid=grid,
        in_specs=[
            pl.BlockSpec((1, 32, 32, C), lambda b: (b, 0, 0, 0)),
            pl.BlockSpec((N_E, E_DIM), lambda b: (0, 0)),
            pl.BlockSpec(memory_space=pltpu.SMEM),
            pl.BlockSpec((C, C), lambda b: (0, 0)),
            pl.BlockSpec((1, C), lambda b: (0, 0)),
        ],
        out_specs=[
            pl.BlockSpec((1, 32, 32, C), lambda b: (b, 0, 0, 0)),
            pl.BlockSpec((1, 1, ROWS_PER_B), lambda b: (b, 0, 0)),
            pl.BlockSpec((1, ROWS_PER_B, N_E), lambda b: (b, 0, 0)),
            pl.BlockSpec((1, 1), lambda b: (0, 0)),
            pl.BlockSpec((1, 1), lambda b: (0, 0)),
        ],
        out_shape=[
            jax.ShapeDtypeStruct((B, 32, 32, C), jnp.float32),
            jax.ShapeDtypeStruct((B, 1, ROWS_PER_B), jnp.int32),
            jax.ShapeDtypeStruct((B, ROWS_PER_B, N_E), jnp.float32),
            jax.ShapeDtypeStruct((1, 1), jnp.float32),
            jax.ShapeDtypeStruct((1, 1), jnp.float32),
        ],
        scratch_shapes=[
            pltpu.VMEM((1, N_E), jnp.float32),
            pltpu.SMEM((1, 1), jnp.float32),
        ],
    )(zt, emb_w, sm, conv_w, cb2)

    out = jnp.transpose(out4, (0, 3, 1, 2))
    loss = loss2.reshape(())
    perplexity = perp2.reshape(())
    min_encodings = enc.reshape(N_ROWS, N_E)
    min_encoding_indices = idx3.reshape(N_ROWS, 1)
    return (out, loss, (perplexity, min_encodings, min_encoding_indices))
